# baseline (speedup 1.0000x reference)
import functools

import jax
import jax.numpy as jnp
import numpy as np
from jax import lax
from jax.experimental import pallas as pl
from jax.experimental.pallas import tpu as pltpu
from jax.experimental.pallas import tpu_sc as plsc

NCLS = 80
CH = 84
CONF = 0.25
N_ANCH = 20000
K = 2000

NW = 32
APW = N_ANCH // NW
ROW = 640
GROUPS = ROW // 16
WORDS = APW * CH + 4
INT_MIN = np.int32(-2**31)


def _sc_scores_body(data_hbm, sc_out, bx_out, buf, srow, brow):
    wid = lax.axis_index("s") * 2 + lax.axis_index("c")
    gstart = wid * (APW * CH)
    astart = (gstart // 8) * 8
    off = gstart - astart
    pltpu.sync_copy(data_hbm.at[pl.ds(astart, WORDS)], buf)
    iota = lax.iota(jnp.int32, 16)

    def group(g, carry):
        anchors = g * 16 + iota
        valid = anchors < APW
        base = off + jnp.minimum(anchors, APW - 1) * CH
        accs = [jnp.full((16,), -jnp.inf, jnp.float32) for _ in range(4)]
        for c in range(4, CH):
            v = plsc.load_gather(buf, [base + c])
            accs[c % 4] = jnp.maximum(accs[c % 4], v)
        s = jnp.maximum(jnp.maximum(accs[0], accs[1]),
                        jnp.maximum(accs[2], accs[3]))
        bsum = plsc.load_gather(buf, [base])
        for c in range(1, 4):
            bsum = bsum + plsc.load_gather(buf, [base + c])
        srow[pl.ds(g * 16, 16)] = jnp.where(valid, s, -jnp.inf)
        brow[pl.ds(g * 16, 16)] = jnp.where(valid, bsum, 0.0)
        return carry

    lax.fori_loop(0, GROUPS, group, jnp.int32(0))
    pltpu.sync_copy(srow, sc_out.at[wid])
    pltpu.sync_copy(brow, bx_out.at[wid])


@functools.cache
def _sc_scores():
    return pl.kernel(
        _sc_scores_body,
        out_type=(jax.ShapeDtypeStruct((NW, ROW), jnp.float32),
                  jax.ShapeDtypeStruct((NW, ROW), jnp.float32)),
        mesh=plsc.VectorSubcoreMesh(core_axis_name="c", subcore_axis_name="s",
                                    num_cores=2, num_subcores=16),
        scratch_types=[pltpu.VMEM((WORDS,), jnp.float32),
                       pltpu.VMEM((ROW,), jnp.float32),
                       pltpu.VMEM((ROW,), jnp.float32)],
        compiler_params=pltpu.CompilerParams(needs_layout_passes=False),
    )


def _tc_select_body(s_ref, b_ref, out_ref):
    s = s_ref[...]
    b = b_ref[...]
    col = lax.broadcasted_iota(jnp.int32, (NW, ROW), 1)
    row = lax.broadcasted_iota(jnp.int32, (NW, ROW), 0)
    valid = col < APW
    bits = lax.bitcast_convert_type(s, jnp.int32)
    key = jnp.where(bits >= 0, bits, (~bits) ^ INT_MIN)
    key = jnp.where(valid, key, INT_MIN)
    idx = jnp.where(valid, row * APW + col, jnp.int32(1 << 30))

    def key_bit(i, p):
        t = p | (jnp.int32(1) << (31 - i))
        cnt = jnp.sum((key >= (t ^ INT_MIN)).astype(jnp.int32))
        return jnp.where(cnt >= K, t, p)

    p_u = lax.fori_loop(0, 32, key_bit, jnp.int32(0))
    t_s = p_u ^ INT_MIN
    cnt_gt = jnp.sum((key > t_s).astype(jnp.int32))
    r = K - cnt_gt
    tie = key == t_s

    def idx_bit(i, q):
        t = q | (jnp.int32(1) << (14 - i))
        c = jnp.sum((tie & (idx < t)).astype(jnp.int32))
        return jnp.where(c < r, t, q)

    m = lax.fori_loop(0, 15, idx_bit, jnp.int32(0))

    beta = jnp.where(p_u < 0, p_u ^ INT_MIN, ~p_u)
    t_f = lax.bitcast_convert_type(beta, jnp.float32)

    sum_gt = jnp.sum(jnp.where(key > t_s, s + b, 0.0))
    sum_tie_b = jnp.sum(jnp.where(tie & (idx <= m), b, 0.0))
    loss_a = sum_gt + r.astype(jnp.float32) * t_f + sum_tie_b
    loss_b = jnp.sum(jnp.where(valid & (s >= CONF), s + b, 0.0))
    out_ref[0, 0] = jnp.where(t_f >= CONF, loss_a, loss_b)


def kernel(data):
    flat = data.reshape(N_ANCH * CH)
    scores, boxsum = _sc_scores()(flat)
    out = pl.pallas_call(
        _tc_select_body,
        out_shape=jax.ShapeDtypeStruct((1, 1), jnp.float32),
        out_specs=pl.BlockSpec(memory_space=pltpu.SMEM),
    )(scores, boxsum)
    return out.reshape(())
